# baseline (speedup 1.0000x reference)
import functools

import jax
import jax.numpy as jnp
from jax import lax
from jax.experimental import pallas as pl
from jax.experimental.pallas import tpu as pltpu
from jax.experimental.pallas import tpu_sc as plsc

B = 16384
D = 128
HD = D // 2
NC = 2
NS = 16
NW = NC * NS
CH = 128
ROWS_PER_W = B // NW
NCHUNK = ROWS_PER_W // CH
BLK = 8192


def _sc_gather_both_packed(u_emb, i_emb, uidx2d, iidx2d):
  mesh = plsc.VectorSubcoreMesh(core_axis_name="c", subcore_axis_name="s")

  @functools.partial(
      pl.kernel,
      mesh=mesh,
      out_type=[
          jax.ShapeDtypeStruct((B, HD), jnp.int32),
          jax.ShapeDtypeStruct((B, HD), jnp.int32),
      ],
      scratch_types=[
          pltpu.VMEM((NCHUNK, CH), jnp.int32),
          pltpu.VMEM((NCHUNK, CH), jnp.int32),
          pltpu.VMEM((CH, D), jnp.float32),
          pltpu.VMEM((CH, D), jnp.float32),
          pltpu.VMEM((CH, D), jnp.float32),
          pltpu.VMEM((CH, D), jnp.float32),
          pltpu.VMEM((CH, HD), jnp.int32),
          pltpu.VMEM((CH, HD), jnp.int32),
          pltpu.SemaphoreType.DMA,
          pltpu.SemaphoreType.DMA,
          pltpu.SemaphoreType.DMA,
          pltpu.SemaphoreType.DMA,
          pltpu.SemaphoreType.DMA,
          pltpu.SemaphoreType.DMA,
      ],
  )
  def k(uemb_hbm, iemb_hbm, uidx_hbm, iidx_hbm, ue_out, ie_out,
        uidx_v, iidx_v, uf0, uf1, if0, if1, ubuf, ibuf,
        ugs0, ugs1, igs0, igs1, uwsem, iwsem):
    wid = lax.axis_index("s") * NC + lax.axis_index("c")
    idx_row0 = wid * NCHUNK
    base = wid * ROWS_PER_W
    pltpu.sync_copy(uidx_hbm.at[pl.ds(idx_row0, NCHUNK)], uidx_v)
    pltpu.sync_copy(iidx_hbm.at[pl.ds(idx_row0, NCHUNK)], iidx_v)
    ufb, ifb = (uf0, uf1), (if0, if1)
    ugs, igs = (ugs0, ugs1), (igs0, igs1)

    himask = jnp.uint32(0xFFFF0000)
    half = jnp.uint32(0x8000)

    def convert(src_f32, dst_i32):
      def row(r, carry):
        for g in range(HD // 16):
          a = lax.bitcast_convert_type(
              src_f32[r, pl.ds(16 * g, 16)], jnp.uint32)
          b = lax.bitcast_convert_type(
              src_f32[r, pl.ds(HD + 16 * g, 16)], jnp.uint32)
          lo = lax.shift_right_logical(a + half, jnp.uint32(16))
          hi = lax.bitwise_and(b + half, himask)
          dst_i32[r, pl.ds(16 * g, 16)] = lax.bitcast_convert_type(
              lax.bitwise_or(lo, hi), jnp.int32)
        return carry
      lax.fori_loop(0, CH, row, 0)

    du = [pltpu.async_copy(uemb_hbm.at[uidx_v.at[j]], ufb[j], ugs[j])
          for j in range(2)]
    di = [pltpu.async_copy(iemb_hbm.at[iidx_v.at[j]], ifb[j], igs[j])
          for j in range(2)]
    wu = None
    wi = None
    for j in range(NCHUNK):
      s = j % 2
      du[s].wait()
      if wu is not None:
        wu.wait()
      convert(ufb[s], ubuf)
      wu = pltpu.async_copy(
          ubuf, ue_out.at[pl.ds(base + j * CH, CH)], uwsem)
      if j + 2 < NCHUNK:
        du[s] = pltpu.async_copy(
            uemb_hbm.at[uidx_v.at[j + 2]], ufb[s], ugs[s])
      di[s].wait()
      if wi is not None:
        wi.wait()
      convert(ifb[s], ibuf)
      wi = pltpu.async_copy(
          ibuf, ie_out.at[pl.ds(base + j * CH, CH)], iwsem)
      if j + 2 < NCHUNK:
        di[s] = pltpu.async_copy(
            iemb_hbm.at[iidx_v.at[j + 2]], ifb[s], igs[s])
    wu.wait()
    wi.wait()

  return k(u_emb, i_emb, uidx2d, iidx2d)


def _split_halves(packed_i32):
  xu = lax.bitcast_convert_type(packed_i32, jnp.uint32)
  lo = lax.bitcast_convert_type(
      lax.shift_left(xu, jnp.uint32(16)), jnp.float32)
  hi = lax.bitcast_convert_type(
      lax.bitwise_and(xu, jnp.uint32(0xFFFF0000)), jnp.float32)
  return lo, hi


def _tc_head_body(ue_ref, ie_ref, w1_ref, b1_ref, w2_ref, b2_ref,
                  w3_ref, b3_ref, out_ref):
  ua, ub = _split_halves(ue_ref[...])
  ia, ib = _split_halves(ie_ref[...])
  w1 = w1_ref[...]
  h1 = jnp.maximum(
      ua @ w1[0:HD] + ub @ w1[HD:D]
      + ia @ w1[D:D + HD] + ib @ w1[D + HD:2 * D]
      + b1_ref[...], 0.0)
  h2 = jnp.maximum(h1 @ w2_ref[...] + b2_ref[...], 0.0)
  mlp = h2 @ w3_ref[...]
  mf = jnp.sum(ua * ia + ub * ib, axis=1, keepdims=True)
  pred_col = mf + mlp + b3_ref[0]
  pred_row = jnp.transpose(pred_col)
  out_ref[...] = jax.nn.sigmoid(pred_row)[None]


def _tc_head(ue, ie, W1, b1r, b2r, W2, W3, b3):
  grid = B // BLK
  out2d = pl.pallas_call(
      _tc_head_body,
      grid=(grid,),
      in_specs=[
          pl.BlockSpec((BLK, HD), lambda i: (i, 0)),
          pl.BlockSpec((BLK, HD), lambda i: (i, 0)),
          pl.BlockSpec((2 * D, D), lambda i: (0, 0)),
          pl.BlockSpec((1, D), lambda i: (0, 0)),
          pl.BlockSpec((D, 16), lambda i: (0, 0)),
          pl.BlockSpec((1, 16), lambda i: (0, 0)),
          pl.BlockSpec((16, 1), lambda i: (0, 0)),
          pl.BlockSpec(memory_space=pltpu.SMEM),
      ],
      out_specs=pl.BlockSpec((1, 1, BLK), lambda i: (i, 0, 0)),
      out_shape=jax.ShapeDtypeStruct((grid, 1, BLK), jnp.float32),
      compiler_params=pltpu.CompilerParams(
          dimension_semantics=("arbitrary",),
      ),
  )(ue, ie, W1, b1r, W2, b2r, W3, b3)
  return out2d.reshape(B)


def kernel(user, item, u_emb, i_emb, W1, b1, W2, b2, W3, b3):
  uidx2d = user[:, 0].reshape(B // CH, CH)
  iidx2d = item[:, 0].reshape(B // CH, CH)
  b1r = b1.reshape(1, D)
  b2r = b2.reshape(1, 16)
  ue, ie = _sc_gather_both_packed(u_emb, i_emb, uidx2d, iidx2d)
  return _tc_head(ue, ie, W1, b1r, b2r, W2, W3, b3)


# speedup vs baseline: 1.0569x; 1.0569x over previous
import functools

import jax
import jax.numpy as jnp
from jax import lax
from jax.experimental import pallas as pl
from jax.experimental.pallas import tpu as pltpu
from jax.experimental.pallas import tpu_sc as plsc

B = 16384
D = 128
NC = 2
NS = 16
NW = NC * NS
CH = 128
ROWS_PER_W = B // NW
NCHUNK = ROWS_PER_W // CH
BLK = 8192


def _sc_gather_both_packed(u_emb, i_emb, uidx2d, iidx2d):
  mesh = plsc.VectorSubcoreMesh(core_axis_name="c", subcore_axis_name="s")

  @functools.partial(
      pl.kernel,
      mesh=mesh,
      out_type=[
          jax.ShapeDtypeStruct((B // 2, D), jnp.int32),
          jax.ShapeDtypeStruct((B // 2, D), jnp.int32),
      ],
      scratch_types=[
          pltpu.VMEM((NCHUNK, CH), jnp.int32),
          pltpu.VMEM((NCHUNK, CH), jnp.int32),
          pltpu.VMEM((CH, D), jnp.float32),
          pltpu.VMEM((CH, D), jnp.float32),
          pltpu.VMEM((CH, D), jnp.float32),
          pltpu.VMEM((CH, D), jnp.float32),
          pltpu.VMEM((CH // 2, D), jnp.int32),
          pltpu.VMEM((CH // 2, D), jnp.int32),
          pltpu.SemaphoreType.DMA,
          pltpu.SemaphoreType.DMA,
          pltpu.SemaphoreType.DMA,
          pltpu.SemaphoreType.DMA,
          pltpu.SemaphoreType.DMA,
          pltpu.SemaphoreType.DMA,
      ],
  )
  def k(uemb_hbm, iemb_hbm, uidx_hbm, iidx_hbm, ue_out, ie_out,
        uidx_v, iidx_v, uf0, uf1, if0, if1, ubuf, ibuf,
        ugs0, ugs1, igs0, igs1, uwsem, iwsem):
    wid = lax.axis_index("s") * NC + lax.axis_index("c")
    idx_row0 = wid * NCHUNK
    base = wid * ROWS_PER_W
    pbase = wid * (ROWS_PER_W // 2)
    pltpu.sync_copy(uidx_hbm.at[pl.ds(idx_row0, NCHUNK)], uidx_v)
    pltpu.sync_copy(iidx_hbm.at[pl.ds(idx_row0, NCHUNK)], iidx_v)
    ufb, ifb = (uf0, uf1), (if0, if1)
    ugs, igs = (ugs0, ugs1), (igs0, igs1)

    himask = jnp.uint32(0xFFFF0000)
    half = jnp.uint32(0x8000)

    def convert(src_f32, dst_i32):
      @functools.partial(plsc.parallel_loop, 0, CH // 2, unroll=4)
      def _(r):
        for g in range(D // 16):
          a = lax.bitcast_convert_type(
              src_f32[2 * r, pl.ds(16 * g, 16)], jnp.uint32)
          b = lax.bitcast_convert_type(
              src_f32[2 * r + 1, pl.ds(16 * g, 16)], jnp.uint32)
          lo = lax.shift_right_logical(a + half, jnp.uint32(16))
          hi = lax.bitwise_and(b + half, himask)
          dst_i32[r, pl.ds(16 * g, 16)] = lax.bitcast_convert_type(
              lax.bitwise_or(lo, hi), jnp.int32)

    du = [pltpu.async_copy(uemb_hbm.at[uidx_v.at[j]], ufb[j], ugs[j])
          for j in range(2)]
    di = [pltpu.async_copy(iemb_hbm.at[iidx_v.at[j]], ifb[j], igs[j])
          for j in range(2)]
    wu = None
    wi = None
    for j in range(NCHUNK):
      s = j % 2
      du[s].wait()
      if wu is not None:
        wu.wait()
      convert(ufb[s], ubuf)
      wu = pltpu.async_copy(
          ubuf, ue_out.at[pl.ds(pbase + j * (CH // 2), CH // 2)], uwsem)
      if j + 2 < NCHUNK:
        du[s] = pltpu.async_copy(
            uemb_hbm.at[uidx_v.at[j + 2]], ufb[s], ugs[s])
      di[s].wait()
      if wi is not None:
        wi.wait()
      convert(ifb[s], ibuf)
      wi = pltpu.async_copy(
          ibuf, ie_out.at[pl.ds(pbase + j * (CH // 2), CH // 2)], iwsem)
      if j + 2 < NCHUNK:
        di[s] = pltpu.async_copy(
            iemb_hbm.at[iidx_v.at[j + 2]], ifb[s], igs[s])
    wu.wait()
    wi.wait()

  return k(u_emb, i_emb, uidx2d, iidx2d)


def _split_rows(packed_i32):
  xu = lax.bitcast_convert_type(packed_i32, jnp.uint32)
  even = lax.bitcast_convert_type(
      lax.shift_left(xu, jnp.uint32(16)), jnp.float32)
  odd = lax.bitcast_convert_type(
      lax.bitwise_and(xu, jnp.uint32(0xFFFF0000)), jnp.float32)
  return even, odd


def _tc_head_body(ue_ref, ie_ref, w1a_ref, w1b_ref, b1_ref, w2_ref, b2_ref,
                  w3_ref, b3_ref, out_ref):
  ue0, ue1 = _split_rows(ue_ref[...])
  ie0, ie1 = _split_rows(ie_ref[...])
  w1a = w1a_ref[...]
  w1b = w1b_ref[...]
  preds = []
  for ue, ie in ((ue0, ie0), (ue1, ie1)):
    h1 = jnp.maximum(ue @ w1a + ie @ w1b + b1_ref[...], 0.0)
    h2 = jnp.maximum(h1 @ w2_ref[...] + b2_ref[...], 0.0)
    mlp = h2 @ w3_ref[...]
    mf = jnp.sum(ue * ie, axis=1, keepdims=True)
    pred_col = mf + mlp + b3_ref[0]
    preds.append(jax.nn.sigmoid(jnp.transpose(pred_col)))
  out_ref[...] = jnp.concatenate(preds, axis=0)[None]


def _tc_head(ue, ie, W1, b1r, b2r, W2, W3, b3):
  grid = B // BLK
  hb = BLK // 2
  out3d = pl.pallas_call(
      _tc_head_body,
      grid=(grid,),
      in_specs=[
          pl.BlockSpec((hb, D), lambda i: (i, 0)),
          pl.BlockSpec((hb, D), lambda i: (i, 0)),
          pl.BlockSpec((D, D), lambda i: (0, 0)),
          pl.BlockSpec((D, D), lambda i: (0, 0)),
          pl.BlockSpec((1, D), lambda i: (0, 0)),
          pl.BlockSpec((D, 16), lambda i: (0, 0)),
          pl.BlockSpec((1, 16), lambda i: (0, 0)),
          pl.BlockSpec((16, 1), lambda i: (0, 0)),
          pl.BlockSpec(memory_space=pltpu.SMEM),
      ],
      out_specs=pl.BlockSpec((1, 2, hb), lambda i: (i, 0, 0)),
      out_shape=jax.ShapeDtypeStruct((grid, 2, hb), jnp.float32),
      compiler_params=pltpu.CompilerParams(
          dimension_semantics=("arbitrary",),
      ),
  )(ue, ie, W1[:D], W1[D:], b1r, W2, b2r, W3, b3)
  return jnp.swapaxes(out3d, 1, 2).reshape(B)


def kernel(user, item, u_emb, i_emb, W1, b1, W2, b2, W3, b3):
  uidx2d = user[:, 0].reshape(B // CH, CH)
  iidx2d = item[:, 0].reshape(B // CH, CH)
  b1r = b1.reshape(1, D)
  b2r = b2.reshape(1, 16)
  ue, ie = _sc_gather_both_packed(u_emb, i_emb, uidx2d, iidx2d)
  return _tc_head(ue, ie, W1, b1r, b2r, W2, W3, b3)


# speedup vs baseline: 1.0997x; 1.0405x over previous
import functools

import jax
import jax.numpy as jnp
from jax import lax
from jax.experimental import pallas as pl
from jax.experimental.pallas import tpu as pltpu
from jax.experimental.pallas import tpu_sc as plsc

B = 16384
D = 128
NC = 2
NS = 16
NW = NC * NS
CH = 128
ROWS_PER_W = B // NW
NCHUNK = ROWS_PER_W // CH
BLK = 8192


def _sc_gather_both(u_emb, i_emb, uidx2d, iidx2d):
  mesh = plsc.VectorSubcoreMesh(core_axis_name="c", subcore_axis_name="s")

  @functools.partial(
      pl.kernel,
      mesh=mesh,
      out_type=[
          jax.ShapeDtypeStruct((B, D), jnp.float32),
          jax.ShapeDtypeStruct((B, D), jnp.float32),
      ],
      scratch_types=[
          pltpu.VMEM((NCHUNK, CH), jnp.int32),
          pltpu.VMEM((NCHUNK, CH), jnp.int32),
          pltpu.VMEM((CH, D), jnp.float32),
          pltpu.VMEM((CH, D), jnp.float32),
          pltpu.VMEM((CH, D), jnp.float32),
          pltpu.VMEM((CH, D), jnp.float32),
          pltpu.SemaphoreType.DMA,
          pltpu.SemaphoreType.DMA,
          pltpu.SemaphoreType.DMA,
          pltpu.SemaphoreType.DMA,
      ],
  )
  def k(uemb_hbm, iemb_hbm, uidx_hbm, iidx_hbm, ue_out, ie_out,
        uidx_v, iidx_v, ubuf0, ubuf1, ibuf0, ibuf1,
        usem0, usem1, isem0, isem1):
    wid = lax.axis_index("s") * NC + lax.axis_index("c")
    idx_row0 = wid * NCHUNK
    base = wid * ROWS_PER_W
    pltpu.sync_copy(uidx_hbm.at[pl.ds(idx_row0, NCHUNK)], uidx_v)
    pltpu.sync_copy(iidx_hbm.at[pl.ds(idx_row0, NCHUNK)], iidx_v)
    ubufs, usems = (ubuf0, ubuf1), (usem0, usem1)
    ibufs, isems = (ibuf0, ibuf1), (isem0, isem1)
    du = [pltpu.async_copy(uemb_hbm.at[uidx_v.at[j]], ubufs[j], usems[j])
          for j in range(2)]
    di = [pltpu.async_copy(iemb_hbm.at[iidx_v.at[j]], ibufs[j], isems[j])
          for j in range(2)]
    for j in range(NCHUNK):
      s = j % 2
      du[s].wait()
      pltpu.sync_copy(ubufs[s], ue_out.at[pl.ds(base + j * CH, CH)])
      if j + 2 < NCHUNK:
        du[s] = pltpu.async_copy(
            uemb_hbm.at[uidx_v.at[j + 2]], ubufs[s], usems[s])
      di[s].wait()
      pltpu.sync_copy(ibufs[s], ie_out.at[pl.ds(base + j * CH, CH)])
      if j + 2 < NCHUNK:
        di[s] = pltpu.async_copy(
            iemb_hbm.at[iidx_v.at[j + 2]], ibufs[s], isems[s])

  return k(u_emb, i_emb, uidx2d, iidx2d)


def _tc_head_body(ue_ref, ie_ref, w1a_ref, w1b_ref, b1_ref, w2_ref, b2_ref,
                  w3_ref, b3_ref, out_ref):
  ue = ue_ref[...]
  ie = ie_ref[...]
  h1 = jnp.maximum(
      ue @ w1a_ref[...] + ie @ w1b_ref[...] + b1_ref[...], 0.0)
  h2 = jnp.maximum(h1 @ w2_ref[...] + b2_ref[...], 0.0)
  mlp = h2 @ w3_ref[...]
  mf = jnp.sum(ue * ie, axis=1, keepdims=True)
  pred_col = mf + mlp + b3_ref[0]
  pred_row = jnp.transpose(pred_col)
  out_ref[...] = jax.nn.sigmoid(pred_row)[None]


def _tc_head(ue, ie, w1a, w1b, b1r, W2, b2r, W3, b3):
  grid = B // BLK
  out2d = pl.pallas_call(
      _tc_head_body,
      grid=(grid,),
      in_specs=[
          pl.BlockSpec((BLK, D), lambda i: (i, 0)),
          pl.BlockSpec((BLK, D), lambda i: (i, 0)),
          pl.BlockSpec((D, D), lambda i: (0, 0)),
          pl.BlockSpec((D, D), lambda i: (0, 0)),
          pl.BlockSpec((1, D), lambda i: (0, 0)),
          pl.BlockSpec((D, 16), lambda i: (0, 0)),
          pl.BlockSpec((1, 16), lambda i: (0, 0)),
          pl.BlockSpec((16, 1), lambda i: (0, 0)),
          pl.BlockSpec(memory_space=pltpu.SMEM),
      ],
      out_specs=pl.BlockSpec((1, 1, BLK), lambda i: (i, 0, 0)),
      out_shape=jax.ShapeDtypeStruct((grid, 1, BLK), jnp.float32),
      compiler_params=pltpu.CompilerParams(
          dimension_semantics=("arbitrary",),
      ),
  )(ue, ie, w1a, w1b, b1r, W2, b2r, W3, b3)
  return out2d.reshape(B)


def kernel(user, item, u_emb, i_emb, W1, b1, W2, b2, W3, b3):
  uidx2d = user[:, 0].reshape(B // CH, CH)
  iidx2d = item[:, 0].reshape(B // CH, CH)
  w1a = W1[:D]
  w1b = W1[D:]
  b1r = b1.reshape(1, D)
  b2r = b2.reshape(1, 16)
  ue, ie = _sc_gather_both(u_emb, i_emb, uidx2d, iidx2d)
  return _tc_head(ue, ie, w1a, w1b, b1r, W2, b2r, W3, b3)
